# baseline (device time: 210096 ns/iter reference)
import jax
import jax.numpy as jnp
from jax import lax
from jax.experimental import pallas as pl
from jax.experimental.pallas import tpu as pltpu

B, S, H, Dh, Dr = 4, 256, 32, 128, 64
D = 4096
DC = 256
DC_SH = 128
SCALE = float((Dh + Dr) ** -0.5)
NBLK = 256
NCH = D // NBLK

_MESH = pl.DeviceIdType.MESH


def _body(x_b_ref, cpeer_ref, wdkv_ref, wuk_ref, wuv_ref, wkr_ref,
          wq_ref, wqr_ref, wo_ref, out_ref,
          q_scr, qr_scr, kr_scr, c_scr, wukf, wuvf, o_scr,
          gs, gr, hs, sy, ry, sx, rx, s2, r2):
    p = pl.program_id(0)
    my_x = lax.axis_index("x")
    my_y = lax.axis_index("y")
    b_mine = 2 * my_x + my_y
    b_y = 2 * my_x + (1 - my_y)
    b_x = 2 * (1 - my_x) + my_y
    ypeer = (my_x, 1 - my_y)
    xpeer = (1 - my_x, my_y)

    my_cols = pl.ds(my_x * DC_SH, DC_SH)
    peer_cols = pl.ds((1 - my_x) * DC_SH, DC_SH)

    def chunk(b, k):
        return out_ref.at[b, :, pl.ds(k * NBLK, NBLK)]

    @pl.when(p == 0)
    def _():
        c_scr[:, my_cols] = jnp.dot(x_b_ref[...], wdkv_ref[...],
                                    preferred_element_type=jnp.float32)
        kr_scr[...] = jnp.dot(x_b_ref[...], wkr_ref[...],
                              preferred_element_type=jnp.float32)
        wukf[my_cols, :] = wuk_ref[...]
        wuvf[my_cols, :] = wuv_ref[...]

        barrier = pltpu.get_barrier_semaphore()
        pl.semaphore_signal(barrier, inc=1, device_id=xpeer,
                            device_id_type=_MESH)
        pl.semaphore_signal(hs.at[0], inc=1, device_id=ypeer,
                            device_id_type=_MESH)
        pl.semaphore_wait(barrier, 1)
        pl.semaphore_wait(hs.at[0], 1)

        pltpu.make_async_remote_copy(
            src_ref=cpeer_ref.at[...],
            dst_ref=c_scr.at[:, my_cols],
            send_sem=gs.at[0], recv_sem=gr.at[0],
            device_id=xpeer, device_id_type=_MESH,
        ).start()
        pltpu.make_async_remote_copy(
            src_ref=wukf.at[my_cols, :],
            dst_ref=wukf.at[my_cols, :],
            send_sem=gs.at[1], recv_sem=gr.at[1],
            device_id=xpeer, device_id_type=_MESH,
        ).start()
        pltpu.make_async_remote_copy(
            src_ref=wuvf.at[my_cols, :],
            dst_ref=wuvf.at[my_cols, :],
            send_sem=gs.at[2], recv_sem=gr.at[2],
            device_id=xpeer, device_id_type=_MESH,
        ).start()

    @pl.when(p < H)
    def _():
        q_scr[:, pl.ds(p * Dh, Dh)] = jnp.dot(
            x_b_ref[...], wq_ref[...], preferred_element_type=jnp.float32)

        @pl.when(p % 2 == 0)
        def _():
            qr_scr[p // 2] = jnp.dot(x_b_ref[...], wqr_ref[...],
                                     preferred_element_type=jnp.float32)

    @pl.when(p == H)
    def _():
        pltpu.make_async_remote_copy(
            src_ref=cpeer_ref.at[...], dst_ref=c_scr.at[:, peer_cols],
            send_sem=gs.at[0], recv_sem=gr.at[0],
            device_id=xpeer, device_id_type=_MESH,
        ).wait_recv()
        pltpu.make_async_remote_copy(
            src_ref=wukf.at[my_cols, :], dst_ref=wukf.at[peer_cols, :],
            send_sem=gs.at[1], recv_sem=gr.at[1],
            device_id=xpeer, device_id_type=_MESH,
        ).wait_recv()
        pltpu.make_async_remote_copy(
            src_ref=wuvf.at[my_cols, :], dst_ref=wuvf.at[peer_cols, :],
            send_sem=gs.at[2], recv_sem=gr.at[2],
            device_id=xpeer, device_id_type=_MESH,
        ).wait_recv()
        pltpu.make_async_remote_copy(
            src_ref=cpeer_ref.at[...], dst_ref=c_scr.at[:, my_cols],
            send_sem=gs.at[0], recv_sem=gr.at[0],
            device_id=xpeer, device_id_type=_MESH,
        ).wait_send()
        pltpu.make_async_remote_copy(
            src_ref=wukf.at[my_cols, :], dst_ref=wukf.at[my_cols, :],
            send_sem=gs.at[1], recv_sem=gr.at[1],
            device_id=xpeer, device_id_type=_MESH,
        ).wait_send()
        pltpu.make_async_remote_copy(
            src_ref=wuvf.at[my_cols, :], dst_ref=wuvf.at[my_cols, :],
            send_sem=gs.at[2], recv_sem=gr.at[2],
            device_id=xpeer, device_id_type=_MESH,
        ).wait_send()

    @pl.when(jnp.logical_and(p >= H, p < 2 * H))
    def _():
        h = p - H
        head_cols = pl.ds(h * Dh, Dh)
        k = jnp.dot(c_scr[...], wukf[:, head_cols],
                    preferred_element_type=jnp.float32)
        v = jnp.dot(c_scr[...], wuvf[:, head_cols],
                    preferred_element_type=jnp.float32)
        q = q_scr[:, head_cols]
        s = lax.dot_general(q, k, (((1,), (1,)), ((), ())),
                            preferred_element_type=jnp.float32)
        qr2 = qr_scr[h // 2]
        lane = lax.broadcasted_iota(jnp.int32, (S, 2 * Dr), 1)
        qr2 = jnp.where((lane // Dr) == (h % 2), qr2, 0.0)
        kr2 = jnp.concatenate([kr_scr[...], kr_scr[...]], axis=1)
        s = s + lax.dot_general(qr2, kr2, (((1,), (1,)), ((), ())),
                                preferred_element_type=jnp.float32)
        s = s * SCALE
        m = jnp.max(s, axis=-1, keepdims=True)
        pr = jnp.exp(s - m)
        pr = pr / jnp.sum(pr, axis=-1, keepdims=True)
        o_scr[:, head_cols] = jnp.dot(pr, v,
                                      preferred_element_type=jnp.float32)

    @pl.when(jnp.logical_and(p >= 2 * H, p < 2 * H + NCH))
    def _():
        n = p - 2 * H
        out_ref[b_mine, :, pl.ds(n * NBLK, NBLK)] = jnp.dot(
            o_scr[...], wo_ref[...], preferred_element_type=jnp.float32)
        pltpu.make_async_remote_copy(
            src_ref=chunk(b_mine, n), dst_ref=chunk(b_mine, n),
            send_sem=sy.at[n], recv_sem=ry.at[n],
            device_id=ypeer, device_id_type=_MESH,
        ).start()
        pltpu.make_async_remote_copy(
            src_ref=chunk(b_mine, n), dst_ref=chunk(b_mine, n),
            send_sem=sx.at[n], recv_sem=rx.at[n],
            device_id=xpeer, device_id_type=_MESH,
        ).start()

    @pl.when(p == 2 * H + NCH)
    def _():
        for k in range(NCH):
            pltpu.make_async_remote_copy(
                src_ref=chunk(b_mine, k), dst_ref=chunk(b_y, k),
                send_sem=sy.at[k], recv_sem=ry.at[k],
                device_id=ypeer, device_id_type=_MESH,
            ).wait_recv()
            pltpu.make_async_remote_copy(
                src_ref=chunk(b_mine, k), dst_ref=chunk(b_x, k),
                send_sem=sx.at[k], recv_sem=rx.at[k],
                device_id=xpeer, device_id_type=_MESH,
            ).wait_recv()
            pltpu.make_async_remote_copy(
                src_ref=chunk(b_mine, k), dst_ref=chunk(b_mine, k),
                send_sem=sy.at[k], recv_sem=ry.at[k],
                device_id=ypeer, device_id_type=_MESH,
            ).wait_send()
            pltpu.make_async_remote_copy(
                src_ref=chunk(b_mine, k), dst_ref=chunk(b_mine, k),
                send_sem=sx.at[k], recv_sem=rx.at[k],
                device_id=xpeer, device_id_type=_MESH,
            ).wait_send()

        r2y = pltpu.make_async_remote_copy(
            src_ref=out_ref.at[b_x, :, pl.ds(0, D // 2)],
            dst_ref=out_ref.at[b_x, :, pl.ds(0, D // 2)],
            send_sem=s2.at[0], recv_sem=r2.at[0],
            device_id=ypeer, device_id_type=_MESH,
        )
        r2x = pltpu.make_async_remote_copy(
            src_ref=out_ref.at[b_y, :, pl.ds(D // 2, D // 2)],
            dst_ref=out_ref.at[b_y, :, pl.ds(D // 2, D // 2)],
            send_sem=s2.at[1], recv_sem=r2.at[1],
            device_id=xpeer, device_id_type=_MESH,
        )
        r2y.start()
        r2x.start()
        r2y.wait()
        r2x.wait()


def _mla(x_b, cpeer, wdkv, wuk, wuv, wkr, wq, wqr, wo):
    return pl.pallas_call(
        _body,
        grid=(2 * H + NCH + 1,),
        in_specs=[
            pl.BlockSpec((S, D), lambda p: (0, 0)),
            pl.BlockSpec((S, DC_SH), lambda p: (0, 0)),
            pl.BlockSpec((D, DC_SH), lambda p: (0, 0)),
            pl.BlockSpec((DC_SH, D), lambda p: (0, 0)),
            pl.BlockSpec((DC_SH, D), lambda p: (0, 0)),
            pl.BlockSpec((D, Dr), lambda p: (0, 0)),
            pl.BlockSpec((D, Dh), lambda p: (0, jnp.minimum(p, H - 1))),
            pl.BlockSpec((D, 2 * Dr), lambda p: (0, jnp.minimum(p, H - 1) // 2)),
            pl.BlockSpec((H * Dh, NBLK),
                         lambda p: (0, jnp.clip(p - 2 * H, 0, NCH - 1))),
        ],
        out_specs=pl.BlockSpec((B, S, D), lambda p: (0, 0, 0)),
        out_shape=jax.ShapeDtypeStruct((B, S, D), jnp.float32),
        scratch_shapes=[
            pltpu.VMEM((S, H * Dh), jnp.float32),
            pltpu.VMEM((H // 2, S, 2 * Dr), jnp.float32),
            pltpu.VMEM((S, Dr), jnp.float32),
            pltpu.VMEM((S, DC), jnp.float32),
            pltpu.VMEM((DC, D), jnp.float32),
            pltpu.VMEM((DC, D), jnp.float32),
            pltpu.VMEM((S, H * Dh), jnp.float32),
            pltpu.SemaphoreType.DMA((3,)),
            pltpu.SemaphoreType.DMA((3,)),
            pltpu.SemaphoreType.REGULAR((1,)),
            pltpu.SemaphoreType.DMA((NCH,)),
            pltpu.SemaphoreType.DMA((NCH,)),
            pltpu.SemaphoreType.DMA((NCH,)),
            pltpu.SemaphoreType.DMA((NCH,)),
            pltpu.SemaphoreType.DMA((2,)),
            pltpu.SemaphoreType.DMA((2,)),
        ],
        compiler_params=pltpu.CompilerParams(
            collective_id=0, vmem_limit_bytes=100 * 1024 * 1024),
    )(x_b, cpeer, wdkv, wuk, wuv, wkr, wq, wqr, wo)


def kernel(x, Wdkv, Wuk, Wuv, Wq, Wqr, Wkr, Wo):
    my_x = lax.axis_index("x")
    my_y = lax.axis_index("y")
    b_mine = 2 * my_x + my_y
    b_xpeer = 2 * (1 - my_x) + my_y
    x_b = lax.dynamic_slice_in_dim(x, b_mine, 1, axis=0)[0]
    x_bp = lax.dynamic_slice_in_dim(x, b_xpeer, 1, axis=0)[0]
    cpeer = x_bp @ Wdkv

    return _mla(x_b, cpeer, Wdkv, Wuk, Wuv, Wkr, Wq, Wqr, Wo)


# device time: 186785 ns/iter; 1.1248x vs baseline; 1.1248x over previous
import jax
import jax.numpy as jnp
from jax import lax
from jax.experimental import pallas as pl
from jax.experimental.pallas import tpu as pltpu

B, S, H, Dh, Dr = 4, 256, 32, 128, 64
D = 4096
DC = 256
DC_SH = 128
SCALE = float((Dh + Dr) ** -0.5)
NBLK = 512
NCH = D // NBLK

_MESH = pl.DeviceIdType.MESH


def _ab_body(x_b_ref, cpeer_ref, wdkv_ref, wuk_ref, wuv_ref, wkr_ref,
             wq_ref, wqr_ref, o_ref,
             q_scr, qr_scr, kr_scr, c_scr, wukf, wuvf,
             send_sems, recv_sems):
    p = pl.program_id(0)
    my_x = lax.axis_index("x")
    my_y = lax.axis_index("y")
    xpeer = (1 - my_x, my_y)

    my_cols = pl.ds(my_x * DC_SH, DC_SH)
    peer_cols = pl.ds((1 - my_x) * DC_SH, DC_SH)

    @pl.when(p == 0)
    def _():
        c_scr[:, my_cols] = jnp.dot(x_b_ref[...], wdkv_ref[...],
                                    preferred_element_type=jnp.float32)
        kr_scr[...] = jnp.dot(x_b_ref[...], wkr_ref[...],
                              preferred_element_type=jnp.float32)
        wukf[my_cols, :] = wuk_ref[...]
        wuvf[my_cols, :] = wuv_ref[...]

        barrier = pltpu.get_barrier_semaphore()
        pl.semaphore_signal(barrier, inc=1, device_id=xpeer,
                            device_id_type=_MESH)
        pl.semaphore_wait(barrier, 1)
        pltpu.make_async_remote_copy(
            src_ref=cpeer_ref.at[...],
            dst_ref=c_scr.at[:, my_cols],
            send_sem=send_sems.at[0], recv_sem=recv_sems.at[0],
            device_id=xpeer, device_id_type=_MESH,
        ).start()
        pltpu.make_async_remote_copy(
            src_ref=wukf.at[my_cols, :],
            dst_ref=wukf.at[my_cols, :],
            send_sem=send_sems.at[1], recv_sem=recv_sems.at[1],
            device_id=xpeer, device_id_type=_MESH,
        ).start()
        pltpu.make_async_remote_copy(
            src_ref=wuvf.at[my_cols, :],
            dst_ref=wuvf.at[my_cols, :],
            send_sem=send_sems.at[2], recv_sem=recv_sems.at[2],
            device_id=xpeer, device_id_type=_MESH,
        ).start()

    @pl.when(p < H)
    def _():
        q_scr[:, pl.ds(p * Dh, Dh)] = jnp.dot(
            x_b_ref[...], wq_ref[...], preferred_element_type=jnp.float32)

        @pl.when(p % 2 == 0)
        def _():
            qr_scr[p // 2] = jnp.dot(x_b_ref[...], wqr_ref[...],
                                     preferred_element_type=jnp.float32)

    @pl.when(p == H)
    def _():
        pltpu.make_async_remote_copy(
            src_ref=cpeer_ref.at[...], dst_ref=c_scr.at[:, peer_cols],
            send_sem=send_sems.at[0], recv_sem=recv_sems.at[0],
            device_id=xpeer, device_id_type=_MESH,
        ).wait_recv()
        pltpu.make_async_remote_copy(
            src_ref=wukf.at[my_cols, :], dst_ref=wukf.at[peer_cols, :],
            send_sem=send_sems.at[1], recv_sem=recv_sems.at[1],
            device_id=xpeer, device_id_type=_MESH,
        ).wait_recv()
        pltpu.make_async_remote_copy(
            src_ref=wuvf.at[my_cols, :], dst_ref=wuvf.at[peer_cols, :],
            send_sem=send_sems.at[2], recv_sem=recv_sems.at[2],
            device_id=xpeer, device_id_type=_MESH,
        ).wait_recv()
        pltpu.make_async_remote_copy(
            src_ref=cpeer_ref.at[...], dst_ref=c_scr.at[:, my_cols],
            send_sem=send_sems.at[0], recv_sem=recv_sems.at[0],
            device_id=xpeer, device_id_type=_MESH,
        ).wait_send()
        pltpu.make_async_remote_copy(
            src_ref=wukf.at[my_cols, :], dst_ref=wukf.at[my_cols, :],
            send_sem=send_sems.at[1], recv_sem=recv_sems.at[1],
            device_id=xpeer, device_id_type=_MESH,
        ).wait_send()
        pltpu.make_async_remote_copy(
            src_ref=wuvf.at[my_cols, :], dst_ref=wuvf.at[my_cols, :],
            send_sem=send_sems.at[2], recv_sem=recv_sems.at[2],
            device_id=xpeer, device_id_type=_MESH,
        ).wait_send()

    @pl.when(p >= H)
    def _():
        h = p - H
        head_cols = pl.ds(h * Dh, Dh)
        k = jnp.dot(c_scr[...], wukf[:, head_cols],
                    preferred_element_type=jnp.float32)
        v = jnp.dot(c_scr[...], wuvf[:, head_cols],
                    preferred_element_type=jnp.float32)
        q = q_scr[:, head_cols]
        s = lax.dot_general(q, k, (((1,), (1,)), ((), ())),
                            preferred_element_type=jnp.float32)
        qr2 = qr_scr[h // 2]
        lane = lax.broadcasted_iota(jnp.int32, (S, 2 * Dr), 1)
        qr2 = jnp.where((lane // Dr) == (h % 2), qr2, 0.0)
        kr2 = jnp.concatenate([kr_scr[...], kr_scr[...]], axis=1)
        s = s + lax.dot_general(qr2, kr2, (((1,), (1,)), ((), ())),
                                preferred_element_type=jnp.float32)
        s = s * SCALE
        m = jnp.max(s, axis=-1, keepdims=True)
        pr = jnp.exp(s - m)
        pr = pr / jnp.sum(pr, axis=-1, keepdims=True)
        o_ref[...] = jnp.dot(pr, v, preferred_element_type=jnp.float32)


def _ab(x_b, cpeer, wdkv, wuk, wuv, wkr, wq, wqr):
    return pl.pallas_call(
        _ab_body,
        grid=(2 * H,),
        in_specs=[
            pl.BlockSpec((S, D), lambda p: (0, 0)),
            pl.BlockSpec((S, DC_SH), lambda p: (0, 0)),
            pl.BlockSpec((D, DC_SH), lambda p: (0, 0)),
            pl.BlockSpec((DC_SH, D), lambda p: (0, 0)),
            pl.BlockSpec((DC_SH, D), lambda p: (0, 0)),
            pl.BlockSpec((D, Dr), lambda p: (0, 0)),
            pl.BlockSpec((D, Dh), lambda p: (0, jnp.minimum(p, H - 1))),
            pl.BlockSpec((D, 2 * Dr), lambda p: (0, jnp.minimum(p, H - 1) // 2)),
        ],
        out_specs=pl.BlockSpec(
            (S, Dh), lambda p: (0, jnp.clip(p - H, 0, H - 1))),
        out_shape=jax.ShapeDtypeStruct((S, H * Dh), jnp.float32),
        scratch_shapes=[
            pltpu.VMEM((S, H * Dh), jnp.float32),
            pltpu.VMEM((H // 2, S, 2 * Dr), jnp.float32),
            pltpu.VMEM((S, Dr), jnp.float32),
            pltpu.VMEM((S, DC), jnp.float32),
            pltpu.VMEM((DC, D), jnp.float32),
            pltpu.VMEM((DC, D), jnp.float32),
            pltpu.SemaphoreType.DMA((3,)),
            pltpu.SemaphoreType.DMA((3,)),
        ],
        compiler_params=pltpu.CompilerParams(collective_id=0),
    )(x_b, cpeer, wdkv, wuk, wuv, wkr, wq, wqr)


def _cd_body(o_ref, wo_ref, out_ref, hs, sy, ry, sx, rx, s2, r2):
    n = pl.program_id(0)
    my_x = lax.axis_index("x")
    my_y = lax.axis_index("y")
    b_mine = 2 * my_x + my_y
    b_y = 2 * my_x + (1 - my_y)
    b_x = 2 * (1 - my_x) + my_y
    ypeer = (my_x, 1 - my_y)
    xpeer = (1 - my_x, my_y)

    def chunk(b, k):
        return out_ref.at[b, :, pl.ds(k * NBLK, NBLK)]

    @pl.when(n == 0)
    def _():
        barrier = pltpu.get_barrier_semaphore()
        for p in (ypeer, xpeer):
            pl.semaphore_signal(barrier, inc=1, device_id=p,
                                device_id_type=_MESH)
        pl.semaphore_wait(barrier, 2)
        pl.semaphore_signal(hs.at[0], inc=1, device_id=ypeer,
                            device_id_type=_MESH)
        pl.semaphore_signal(hs.at[1], inc=1, device_id=xpeer,
                            device_id_type=_MESH)
        pl.semaphore_wait(hs.at[0], 1)
        pl.semaphore_wait(hs.at[1], 1)

    @pl.when(n < NCH)
    def _():
        out_ref[b_mine, :, pl.ds(n * NBLK, NBLK)] = jnp.dot(
            o_ref[...], wo_ref[...], preferred_element_type=jnp.float32)
        pltpu.make_async_remote_copy(
            src_ref=chunk(b_mine, n), dst_ref=chunk(b_mine, n),
            send_sem=sy.at[n], recv_sem=ry.at[n],
            device_id=ypeer, device_id_type=_MESH,
        ).start()
        pltpu.make_async_remote_copy(
            src_ref=chunk(b_mine, n), dst_ref=chunk(b_mine, n),
            send_sem=sx.at[n], recv_sem=rx.at[n],
            device_id=xpeer, device_id_type=_MESH,
        ).start()

    @pl.when(n == NCH)
    def _():
        for k in range(NCH):
            pltpu.make_async_remote_copy(
                src_ref=chunk(b_mine, k), dst_ref=chunk(b_y, k),
                send_sem=sy.at[k], recv_sem=ry.at[k],
                device_id=ypeer, device_id_type=_MESH,
            ).wait_recv()
            pltpu.make_async_remote_copy(
                src_ref=chunk(b_mine, k), dst_ref=chunk(b_x, k),
                send_sem=sx.at[k], recv_sem=rx.at[k],
                device_id=xpeer, device_id_type=_MESH,
            ).wait_recv()
            pltpu.make_async_remote_copy(
                src_ref=chunk(b_mine, k), dst_ref=chunk(b_mine, k),
                send_sem=sy.at[k], recv_sem=ry.at[k],
                device_id=ypeer, device_id_type=_MESH,
            ).wait_send()
            pltpu.make_async_remote_copy(
                src_ref=chunk(b_mine, k), dst_ref=chunk(b_mine, k),
                send_sem=sx.at[k], recv_sem=rx.at[k],
                device_id=xpeer, device_id_type=_MESH,
            ).wait_send()

        r2y = pltpu.make_async_remote_copy(
            src_ref=out_ref.at[b_x, :, pl.ds(0, D // 2)],
            dst_ref=out_ref.at[b_x, :, pl.ds(0, D // 2)],
            send_sem=s2.at[0], recv_sem=r2.at[0],
            device_id=ypeer, device_id_type=_MESH,
        )
        r2x = pltpu.make_async_remote_copy(
            src_ref=out_ref.at[b_y, :, pl.ds(D // 2, D // 2)],
            dst_ref=out_ref.at[b_y, :, pl.ds(D // 2, D // 2)],
            send_sem=s2.at[1], recv_sem=r2.at[1],
            device_id=xpeer, device_id_type=_MESH,
        )
        r2y.start()
        r2x.start()
        r2y.wait()
        r2x.wait()


def _cd(o, wo):
    return pl.pallas_call(
        _cd_body,
        grid=(NCH + 1,),
        in_specs=[
            pl.BlockSpec((S, H * Dh), lambda n: (0, 0)),
            pl.BlockSpec((H * Dh, NBLK), lambda n: (0, jnp.minimum(n, NCH - 1))),
        ],
        out_specs=pl.BlockSpec((B, S, D), lambda n: (0, 0, 0)),
        out_shape=jax.ShapeDtypeStruct((B, S, D), jnp.float32),
        scratch_shapes=[
            pltpu.SemaphoreType.REGULAR((2,)),
            pltpu.SemaphoreType.DMA((NCH,)),
            pltpu.SemaphoreType.DMA((NCH,)),
            pltpu.SemaphoreType.DMA((NCH,)),
            pltpu.SemaphoreType.DMA((NCH,)),
            pltpu.SemaphoreType.DMA((2,)),
            pltpu.SemaphoreType.DMA((2,)),
        ],
        compiler_params=pltpu.CompilerParams(collective_id=1),
    )(o, wo)


def kernel(x, Wdkv, Wuk, Wuv, Wq, Wqr, Wkr, Wo):
    my_x = lax.axis_index("x")
    my_y = lax.axis_index("y")
    b_mine = 2 * my_x + my_y
    b_xpeer = 2 * (1 - my_x) + my_y
    x_b = lax.dynamic_slice_in_dim(x, b_mine, 1, axis=0)[0]
    x_bp = lax.dynamic_slice_in_dim(x, b_xpeer, 1, axis=0)[0]
    cpeer = x_bp @ Wdkv

    o = _ab(x_b, cpeer, Wdkv, Wuk, Wuv, Wkr, Wq, Wqr)
    return _cd(o, Wo)
